# baseline (speedup 1.0000x reference)
import jax
import jax.numpy as jnp
from jax.experimental import pallas as pl

_NUM_CLASSES = 1000
_ROWS = 1024 * 26
_BR = 2048
_NB = _ROWS // _BR


def _body(x_ref, o_ref):
    xv = x_ref[0, 0, :]
    col = jax.lax.broadcasted_iota(jnp.int32, (_BR, _NUM_CLASSES), 1)
    o_ref[...] = (col == xv[:, None]).astype(jnp.float32)


def kernel(x):
    xf = x.astype(jnp.int32).reshape(_NB, 1, _BR)
    y = pl.pallas_call(
        _body,
        grid=(_NB,),
        in_specs=[pl.BlockSpec((1, 1, _BR), lambda i: (i, 0, 0))],
        out_specs=pl.BlockSpec((_BR, _NUM_CLASSES), lambda i: (i, 0)),
        out_shape=jax.ShapeDtypeStruct((_ROWS, _NUM_CLASSES), jnp.float32),
    )(xf)
    return y.reshape(1024, 26, _NUM_CLASSES)


# speedup vs baseline: 1.4037x; 1.4037x over previous
import jax
import jax.numpy as jnp
from jax.experimental import pallas as pl

_NUM_CLASSES = 1000
_B0 = 64
_NB = 1024 // _B0


def _body(x_ref, o_ref):
    xv = x_ref[...]
    col = jax.lax.broadcasted_iota(jnp.int32, (_B0, 26, _NUM_CLASSES), 2)
    o_ref[...] = (col == xv[:, :, None]).astype(jnp.float32)


def kernel(x):
    return pl.pallas_call(
        _body,
        grid=(_NB,),
        in_specs=[pl.BlockSpec((_B0, 26), lambda i: (i, 0))],
        out_specs=pl.BlockSpec((_B0, 26, _NUM_CLASSES), lambda i: (i, 0, 0)),
        out_shape=jax.ShapeDtypeStruct((1024, 26, _NUM_CLASSES), jnp.float32),
    )(x.astype(jnp.int32))


# speedup vs baseline: 6.6596x; 4.7444x over previous
import jax
import jax.numpy as jnp
from jax.experimental import pallas as pl

_NUM_CLASSES = 1000
_N = 1024


def _body(x_ref, o_ref):
    row = jax.lax.broadcasted_iota(jnp.int32, (1, _NUM_CLASSES, _N), 1)
    o_ref[...] = (row == x_ref[...]).astype(jnp.float32)


def kernel(x):
    xt = x.astype(jnp.int32).T.reshape(26, 1, _N)
    yt = pl.pallas_call(
        _body,
        grid=(26,),
        in_specs=[pl.BlockSpec((1, 1, _N), lambda j: (j, 0, 0))],
        out_specs=pl.BlockSpec((1, _NUM_CLASSES, _N), lambda j: (j, 0, 0)),
        out_shape=jax.ShapeDtypeStruct((26, _NUM_CLASSES, _N), jnp.float32),
    )(xt)
    return jnp.transpose(yt, (2, 0, 1))
